# baseline (speedup 1.0000x reference)
import jax
import jax.numpy as jnp
from jax import lax
from jax.experimental import pallas as pl
from jax.experimental.pallas import tpu as pltpu, tpu_sc as plsc

_J = 3
_NW = 4
_D = 3
_N = 2048
_LANES = 16
_NSUB = 16
_W = 80
_NVREG = _W // _LANES
_PTS = _N // _NSUB
_CHUNK = _PTS * _W


def _sc_body(pc_hbm, a_hbm, out_hbm, slab_v, acc_v, part_sh, part_v, a_v, out_v):
    c = lax.axis_index("c")
    s = lax.axis_index("s")

    @pl.when(c == 0)
    def _reduce_slab():
        pltpu.sync_copy(pc_hbm.at[pl.ds(s * _CHUNK, _CHUNK)], slab_v)

        def step(i, accs):
            return tuple(
                accs[g] + slab_v[pl.ds(i * _W + g * _LANES, _LANES)]
                for g in range(_NVREG)
            )

        accs = lax.fori_loop(
            0,
            _PTS,
            step,
            tuple(jnp.zeros((_LANES,), jnp.float32) for _ in range(_NVREG)),
        )
        for g in range(_NVREG):
            acc_v[pl.ds(g * _LANES, _LANES)] = accs[g]
        pltpu.sync_copy(acc_v, part_sh.at[pl.ds(s * _W, _W)])

    plsc.subcore_barrier()

    pltpu.sync_copy(part_sh, part_v)
    pltpu.sync_copy(a_hbm, a_v)
    inv_n = 1.0 / _N
    for g in range(_NVREG):
        m_vec = jnp.zeros((_LANES,), jnp.float32)
        for w in range(_NSUB):
            m_vec = m_vec + part_v[pl.ds(w * _W + g * _LANES, _LANES)]
        out_v[pl.ds(g * _LANES, _LANES)] = (
            m_vec * a_v[pl.ds(g * _LANES, _LANES)] * inv_n
        )

    @pl.when(jnp.logical_and(c == 0, s == 0))
    def _publish():
        pltpu.sync_copy(out_v, out_hbm)


def kernel(point_clouds, sigma, alphas):
    del sigma
    b_pc, n, d = point_clouds.shape
    per_pc = _NW * _J * _D
    n_out = b_pc * per_pc
    cmap = [(cc // per_pc) * d + cc % d for cc in range(n_out)] + [0] * (_W - n_out)
    pc_t = point_clouds.transpose(1, 0, 2).reshape(n, b_pc * d)
    pc_wide = jnp.take(pc_t, jnp.array(cmap, dtype=jnp.int32), axis=1).reshape(-1)
    a36 = jnp.tile(alphas[:, None, :], (1, _J, 1)).reshape(per_pc)
    a80 = jnp.concatenate(
        [jnp.tile(a36, (b_pc,)), jnp.zeros((_W - n_out,), jnp.float32)]
    )
    mesh = plsc.VectorSubcoreMesh(
        core_axis_name="c", subcore_axis_name="s", num_cores=2, num_subcores=_NSUB
    )
    out80 = pl.kernel(
        _sc_body,
        out_type=jax.ShapeDtypeStruct((_W,), jnp.float32),
        mesh=mesh,
        scratch_types=[
            pltpu.VMEM((_CHUNK,), jnp.float32),
            pltpu.VMEM((_W,), jnp.float32),
            pltpu.VMEM_SHARED((_NSUB * _W,), jnp.float32),
            pltpu.VMEM((_NSUB * _W,), jnp.float32),
            pltpu.VMEM((_W,), jnp.float32),
            pltpu.VMEM((_W,), jnp.float32),
        ],
    )(pc_wide, a80)
    return out80[:n_out].reshape(b_pc, per_pc)


# speedup vs baseline: 1.0808x; 1.0808x over previous
import jax
import jax.numpy as jnp
from jax import lax
from jax.experimental import pallas as pl
from jax.experimental.pallas import tpu as pltpu, tpu_sc as plsc

_J = 3
_NW = 4
_D = 3
_N = 2048
_LANES = 16
_NSUB = 16
_W = 32
_NVREG = _W // _LANES
_PTS = _N // _NSUB
_CHUNK = _PTS * _W


def _sc_body(pc_hbm, a_hbm, out_hbm, slab_v, acc_v, part_sh, part_v, a_v, out_v):
    c = lax.axis_index("c")
    s = lax.axis_index("s")
    tile0 = jnp.logical_and(c == 0, s == 0)

    @pl.when(tile0)
    def _prefetch_alphas():
        pltpu.sync_copy(a_hbm, a_v)

    @pl.when(c == 0)
    def _reduce_slab():
        pltpu.sync_copy(pc_hbm.at[pl.ds(s * _CHUNK, _CHUNK)], slab_v)

        def step(i, accs):
            return tuple(
                accs[g] + slab_v[pl.ds(i * _W + g * _LANES, _LANES)]
                for g in range(_NVREG)
            )

        accs = lax.fori_loop(
            0,
            _PTS,
            step,
            tuple(jnp.zeros((_LANES,), jnp.float32) for _ in range(_NVREG)),
        )
        for g in range(_NVREG):
            acc_v[pl.ds(g * _LANES, _LANES)] = accs[g]
        pltpu.sync_copy(acc_v, part_sh.at[pl.ds(s * _W, _W)])

    plsc.subcore_barrier()

    @pl.when(tile0)
    def _finalize():
        pltpu.sync_copy(part_sh, part_v)
        inv_n = 1.0 / _N
        for g in range(_NVREG):
            m_vec = jnp.zeros((_LANES,), jnp.float32)
            for w in range(_NSUB):
                m_vec = m_vec + part_v[pl.ds(w * _W + g * _LANES, _LANES)]
            out_v[pl.ds(g * _LANES, _LANES)] = (
                m_vec * a_v[pl.ds(g * _LANES, _LANES)] * inv_n
            )
        pltpu.sync_copy(out_v, out_hbm)


def kernel(point_clouds, sigma, alphas):
    del sigma
    b_pc, n, d = point_clouds.shape
    n_dist = b_pc * _NW * d
    cmap = [(cc // (_NW * d)) * d + cc % d for cc in range(n_dist)] + [0] * (
        _W - n_dist
    )
    pc_t = point_clouds.transpose(1, 0, 2).reshape(n, b_pc * d)
    pc_wide = jnp.take(pc_t, jnp.array(cmap, dtype=jnp.int32), axis=1).reshape(-1)
    a_wide = jnp.concatenate(
        [
            jnp.tile(alphas.reshape(_NW * d), (b_pc,)),
            jnp.zeros((_W - n_dist,), jnp.float32),
        ]
    )
    mesh = plsc.VectorSubcoreMesh(
        core_axis_name="c", subcore_axis_name="s", num_cores=2, num_subcores=_NSUB
    )
    out32 = pl.kernel(
        _sc_body,
        out_type=jax.ShapeDtypeStruct((_W,), jnp.float32),
        mesh=mesh,
        scratch_types=[
            pltpu.VMEM((_CHUNK,), jnp.float32),
            pltpu.VMEM((_W,), jnp.float32),
            pltpu.VMEM_SHARED((_NSUB * _W,), jnp.float32),
            pltpu.VMEM((_NSUB * _W,), jnp.float32),
            pltpu.VMEM((_W,), jnp.float32),
            pltpu.VMEM((_W,), jnp.float32),
        ],
    )(pc_wide, a_wide)
    v = out32[:n_dist].reshape(b_pc, _NW, 1, d)
    return jnp.tile(v, (1, 1, _J, 1)).reshape(b_pc, _NW * _J * _D)


# speedup vs baseline: 1.1519x; 1.0658x over previous
import jax
import jax.numpy as jnp
from jax import lax
from jax.experimental import pallas as pl
from jax.experimental.pallas import tpu as pltpu, tpu_sc as plsc

_J = 3
_NW = 4
_D = 3
_N = 2048
_LANES = 16
_NSUB = 16
_W = 32
_NVREG = _W // _LANES
_PTS = _N // _NSUB
_CHUNK = _PTS * _W


def _sc_body(pc_hbm, a_hbm, out_hbm, slab_v, acc_v, part_sh, part_v, a_v, out_v):
    c = lax.axis_index("c")
    s = lax.axis_index("s")
    tile0 = jnp.logical_and(c == 0, s == 0)

    @pl.when(tile0)
    def _prefetch_alphas():
        pltpu.sync_copy(a_hbm, a_v)

    @pl.when(c == 0)
    def _reduce_slab():
        pltpu.sync_copy(pc_hbm.at[pl.ds(s * _CHUNK, _CHUNK)], slab_v)

        def step(i, accs):
            return tuple(
                accs[g] + slab_v[pl.ds(i * _W + g * _LANES, _LANES)]
                for g in range(_NVREG)
            )

        accs = lax.fori_loop(
            0,
            _PTS,
            step,
            tuple(jnp.zeros((_LANES,), jnp.float32) for _ in range(_NVREG)),
        )
        for g in range(_NVREG):
            acc_v[pl.ds(g * _LANES, _LANES)] = accs[g]
        pltpu.sync_copy(acc_v, part_sh.at[pl.ds(s * _W, _W)])

    plsc.subcore_barrier()

    @pl.when(tile0)
    def _finalize():
        pltpu.sync_copy(part_sh, part_v)
        inv_n = 1.0 / _N
        for g in range(_NVREG):
            m_vec = jnp.zeros((_LANES,), jnp.float32)
            for w in range(_NSUB):
                m_vec = m_vec + part_v[pl.ds(w * _W + g * _LANES, _LANES)]
            out_v[pl.ds(g * _LANES, _LANES)] = (
                m_vec * a_v[pl.ds(g * _LANES, _LANES)] * inv_n
            )
        pltpu.sync_copy(out_v, out_hbm)


def kernel(point_clouds, sigma, alphas):
    del sigma
    b_pc, n, d = point_clouds.shape
    n_dist = b_pc * _NW * d
    cmap = [(cc // (_NW * d)) * d + cc % d for cc in range(n_dist)] + [0] * (
        _W - n_dist
    )
    pc_t = point_clouds.transpose(1, 0, 2).reshape(n, b_pc * d)
    pc_wide = jnp.take(pc_t, jnp.array(cmap, dtype=jnp.int32), axis=1).reshape(-1)
    a_wide = jnp.concatenate(
        [
            jnp.tile(alphas.reshape(_NW * d), (b_pc,)),
            jnp.zeros((_W - n_dist,), jnp.float32),
        ]
    )
    mesh = plsc.VectorSubcoreMesh(
        core_axis_name="c", subcore_axis_name="s", num_cores=1, num_subcores=_NSUB
    )
    out32 = pl.kernel(
        _sc_body,
        out_type=jax.ShapeDtypeStruct((_W,), jnp.float32),
        mesh=mesh,
        scratch_types=[
            pltpu.VMEM((_CHUNK,), jnp.float32),
            pltpu.VMEM((_W,), jnp.float32),
            pltpu.VMEM_SHARED((_NSUB * _W,), jnp.float32),
            pltpu.VMEM((_NSUB * _W,), jnp.float32),
            pltpu.VMEM((_W,), jnp.float32),
            pltpu.VMEM((_W,), jnp.float32),
        ],
    )(pc_wide, a_wide)
    v = out32[:n_dist].reshape(b_pc, _NW, 1, d)
    return jnp.tile(v, (1, 1, _J, 1)).reshape(b_pc, _NW * _J * _D)


# speedup vs baseline: 4.9497x; 4.2971x over previous
import jax
import jax.numpy as jnp
from jax import lax
from jax.experimental import pallas as pl

_J = 3
_NW = 4
_D = 3


def _body(pc_ref, a_ref, out_ref):
    cols = _NW * _J * _D
    x = pc_ref[...]
    s = jnp.sum(x.reshape(2, 16, 3, 128), axis=1)
    acc = jnp.zeros((2, cols), dtype=jnp.float32)
    for a in range(3):
        l_idx = lax.broadcasted_iota(jnp.int32, (128, cols), 0)
        c_idx = lax.broadcasted_iota(jnp.int32, (128, cols), 1)
        sel = ((2 * a + l_idx) % 3 == c_idx % 3).astype(jnp.float32)
        acc = acc + jnp.dot(s[:, a, :], sel, preferred_element_type=jnp.float32)
    m_exp = acc * (1.0 / 2048.0)
    al = a_ref[...]
    i_idx = lax.broadcasted_iota(jnp.int32, (_NW, cols), 0)
    c_i = lax.broadcasted_iota(jnp.int32, (_NW, cols), 1)
    r_sel = (i_idx == c_i // (_J * _D)).astype(jnp.float32)
    k_idx = lax.broadcasted_iota(jnp.int32, (_D, cols), 0)
    c_k = lax.broadcasted_iota(jnp.int32, (_D, cols), 1)
    k_sel = (k_idx == c_k % _D).astype(jnp.float32)
    a_exp = lax.dot_general(
        al, r_sel, (((0,), (0,)), ((), ())), preferred_element_type=jnp.float32
    )
    a36 = jnp.sum(a_exp * k_sel, axis=0, keepdims=True)
    out_ref[...] = m_exp * a36


def kernel(point_clouds, sigma, alphas):
    del sigma
    b_pc = point_clouds.shape[0]
    pc_flat = point_clouds.reshape(96, 128)
    return pl.pallas_call(
        _body,
        out_shape=jax.ShapeDtypeStruct((b_pc, _NW * _J * _D), jnp.float32),
    )(pc_flat, alphas)


# speedup vs baseline: 5.2398x; 1.0586x over previous
import jax
import jax.numpy as jnp
from jax import lax
from jax.experimental import pallas as pl

_J = 3
_NW = 4
_D = 3


def _body(pc_ref, a_ref, out_ref):
    cols = _NW * _J * _D
    x = pc_ref[...]
    s = jnp.sum(x.reshape(2, 4, 1536), axis=1)
    l_idx = lax.broadcasted_iota(jnp.int32, (1536, cols), 0)
    c_idx = lax.broadcasted_iota(jnp.int32, (1536, cols), 1)
    sel = (l_idx % 3 == c_idx % 3).astype(jnp.float32)
    m_exp = jnp.dot(s, sel, preferred_element_type=jnp.float32) * (1.0 / 2048.0)
    al = a_ref[...]
    i_idx = lax.broadcasted_iota(jnp.int32, (_NW, cols), 0)
    c_i = lax.broadcasted_iota(jnp.int32, (_NW, cols), 1)
    r_sel = (i_idx == c_i // (_J * _D)).astype(jnp.float32)
    k_idx = lax.broadcasted_iota(jnp.int32, (_D, cols), 0)
    c_k = lax.broadcasted_iota(jnp.int32, (_D, cols), 1)
    k_sel = (k_idx == c_k % _D).astype(jnp.float32)
    a_exp = lax.dot_general(
        al, r_sel, (((0,), (0,)), ((), ())), preferred_element_type=jnp.float32
    )
    a36 = jnp.sum(a_exp * k_sel, axis=0, keepdims=True)
    out_ref[...] = m_exp * a36


def kernel(point_clouds, sigma, alphas):
    del sigma
    b_pc = point_clouds.shape[0]
    pc_flat = point_clouds.reshape(8, 1536)
    return pl.pallas_call(
        _body,
        out_shape=jax.ShapeDtypeStruct((b_pc, _NW * _J * _D), jnp.float32),
    )(pc_flat, alphas)
